# Initial kernel scaffold; baseline (speedup 1.0000x reference)
import jax
import jax.numpy as jnp
from jax.experimental import pallas as pl


def kernel(batch, labels):
    raise NotImplementedError("write your pallas kernel here")


# speedup vs baseline: 34.8807x; 34.8807x over previous
import functools

import jax
import jax.numpy as jnp
from jax.experimental import pallas as pl

_K = 5
_BIG = 3.0e38


def _knn_loss_kernel(x_row_ref, x_full_ref, lab_row_ref, lab_col_ref, out_ref,
                     *, n_total, block_rows, num_blocks):
    i = pl.program_id(0)
    x = x_row_ref[...]
    xf = x_full_ref[...]

    sq_r = jnp.sum(x * x, axis=1, keepdims=True)
    ones = jnp.ones((1, x.shape[1]), dtype=jnp.float32)
    sq_c = jax.lax.dot_general(ones, xf * xf,
                               (((1,), (1,)), ((), ())),
                               preferred_element_type=jnp.float32)

    dot = jax.lax.dot_general(x, xf, (((1,), (1,)), ((), ())),
                              preferred_element_type=jnp.float32)
    d2 = sq_r + sq_c - 2.0 * dot
    dist = jnp.sqrt(jnp.maximum(d2, 0.0))

    row_ids = i * block_rows + jax.lax.broadcasted_iota(
        jnp.int32, dist.shape, 0)
    col_ids = jax.lax.broadcasted_iota(jnp.int32, dist.shape, 1)
    dist = jnp.where(row_ids == col_ids, _BIG, dist)

    match = (lab_row_ref[...] == lab_col_ref[...])

    d0 = None
    sum_md = jnp.zeros_like(sq_r)
    sum_m = jnp.zeros_like(sq_r)
    sum_e = jnp.zeros_like(sq_r)
    for _ in range(_K):
        mn = jnp.min(dist, axis=1, keepdims=True)
        first = jnp.min(jnp.where(dist == mn, col_ids, n_total),
                        axis=1, keepdims=True)
        sel = col_ids == first
        mk = jnp.sum(jnp.where(sel & match, 1.0, 0.0),
                     axis=1, keepdims=True)
        if d0 is None:
            d0 = mn
        sum_md += mk * mn
        sum_m += mk
        sum_e += jnp.exp(d0 - mn)
        dist = jnp.where(sel, _BIG, dist)

    lse = jnp.log(sum_e) - d0
    block_sum = jnp.sum(sum_md + sum_m * lse).reshape(1, 1)

    @pl.when(i == 0)
    def _init():
        out_ref[...] = jnp.zeros((1, 1), jnp.float32)

    out_ref[...] += block_sum

    @pl.when(i == num_blocks - 1)
    def _finish():
        out_ref[...] = out_ref[...] / n_total


def kernel(batch, labels):
    n, d = batch.shape
    block_rows = 256
    num_blocks = n // block_rows
    lab_row = labels.reshape(n, 1)
    lab_col = labels.reshape(1, n)

    body = functools.partial(_knn_loss_kernel, n_total=n,
                             block_rows=block_rows, num_blocks=num_blocks)
    out = pl.pallas_call(
        body,
        grid=(num_blocks,),
        in_specs=[
            pl.BlockSpec((block_rows, d), lambda i: (i, 0)),
            pl.BlockSpec((n, d), lambda i: (0, 0)),
            pl.BlockSpec((block_rows, 1), lambda i: (i, 0)),
            pl.BlockSpec((1, n), lambda i: (0, 0)),
        ],
        out_specs=pl.BlockSpec((1, 1), lambda i: (0, 0)),
        out_shape=jax.ShapeDtypeStruct((1, 1), jnp.float32),
    )(batch, batch, lab_row, lab_col)
    return out[0, 0]


# speedup vs baseline: 49.0426x; 1.4060x over previous
import functools

import jax
import jax.numpy as jnp
from jax.experimental import pallas as pl

_K = 5
_BIG = 3.0e38
_BIGI = 1 << 30


def _knn_loss_kernel(x_row_ref, x_full_ref, lab_row_ref, lab_col_ref, out_ref,
                     *, n_total, block_rows, num_blocks):
    i = pl.program_id(0)
    x = x_row_ref[...]
    xf = x_full_ref[...]

    sq_r = jnp.sum(x * x, axis=1, keepdims=True)
    ones = jnp.ones((1, x.shape[1]), dtype=jnp.float32)
    sq_c = jax.lax.dot_general(ones, xf * xf,
                               (((1,), (1,)), ((), ())),
                               preferred_element_type=jnp.float32)

    dot = jax.lax.dot_general(x, xf, (((1,), (1,)), ((), ())),
                              preferred_element_type=jnp.float32)
    d2 = jnp.maximum(sq_r + sq_c - 2.0 * dot, 0.0)

    row_ids = i * block_rows + jax.lax.broadcasted_iota(
        jnp.int32, d2.shape, 0)
    col_ids = jax.lax.broadcasted_iota(jnp.int32, d2.shape, 1)
    d2 = jnp.where(row_ids == col_ids, _BIG, d2)

    match = lab_row_ref[...] == lab_col_ref[...]
    mcol = 2 * col_ids + 1 - match.astype(jnp.int32)

    d0 = None
    sum_md = jnp.zeros_like(sq_r)
    sum_m = jnp.zeros_like(sq_r)
    sum_e = jnp.zeros_like(sq_r)
    mn2 = jnp.min(d2, axis=1, keepdims=True)
    for k in range(_K):
        first = jnp.min(jnp.where(d2 == mn2, mcol, _BIGI),
                        axis=1, keepdims=True)
        mk = (1 - (first & 1)).astype(jnp.float32)
        dk = jnp.sqrt(mn2)
        if d0 is None:
            d0 = dk
        sum_md += mk * dk
        sum_m += mk
        sum_e += jnp.exp(d0 - dk)
        if k < _K - 1:
            d2 = jnp.where(mcol == first, _BIG, d2)
            mn2 = jnp.min(d2, axis=1, keepdims=True)

    lse = jnp.log(sum_e) - d0
    block_sum = jnp.sum(sum_md + sum_m * lse).reshape(1, 1)

    @pl.when(i == 0)
    def _init():
        out_ref[...] = jnp.zeros((1, 1), jnp.float32)

    out_ref[...] += block_sum

    @pl.when(i == num_blocks - 1)
    def _finish():
        out_ref[...] = out_ref[...] / n_total


def kernel(batch, labels):
    n, d = batch.shape
    block_rows = 256
    num_blocks = n // block_rows
    lab_row = labels.reshape(n, 1)
    lab_col = labels.reshape(1, n)

    body = functools.partial(_knn_loss_kernel, n_total=n,
                             block_rows=block_rows, num_blocks=num_blocks)
    out = pl.pallas_call(
        body,
        grid=(num_blocks,),
        in_specs=[
            pl.BlockSpec((block_rows, d), lambda i: (i, 0)),
            pl.BlockSpec((n, d), lambda i: (0, 0)),
            pl.BlockSpec((block_rows, 1), lambda i: (i, 0)),
            pl.BlockSpec((1, n), lambda i: (0, 0)),
        ],
        out_specs=pl.BlockSpec((1, 1), lambda i: (0, 0)),
        out_shape=jax.ShapeDtypeStruct((1, 1), jnp.float32),
    )(batch, batch, lab_row, lab_col)
    return out[0, 0]
